# baseline (speedup 1.0000x reference)
import jax
import jax.numpy as jnp
from jax import lax
from jax.experimental import pallas as pl
from jax.experimental.pallas import tpu as pltpu
from jax.experimental.pallas import tpu_sc as plsc

_N, _D, _S = 100000, 512, 20000
_BNG = 2000
_NW = 32
_NB = 32
_NBP = 48
_BLKS = 98
_BS = 2000
_NEG = -3.0e38


def _gate_body(x_ref, w1_ref, b1_ref, w2_ref, b2_ref, gate_ref, m_ref, macc):
    i = pl.program_id(0)
    h = jnp.maximum(
        jnp.dot(x_ref[...], w1_ref[...], preferred_element_type=jnp.float32)
        + b1_ref[...], 0.0)
    g = jnp.sum(h * w2_ref[...], axis=1, keepdims=True) + b2_ref[0, 0]
    gate_ref[...] = g
    bm = jnp.max(g)
    prev = jnp.where(i == 0, _NEG, macc[0])
    cur = jnp.maximum(prev, bm)
    macc[0] = cur

    @pl.when(i == pl.num_programs(0) - 1)
    def _():
        m_ref[0, 0] = cur


def _gate_call(x, w1, b1, w2, b2):
    return pl.pallas_call(
        _gate_body,
        grid=(_N // _BNG,),
        in_specs=[
            pl.BlockSpec((_BNG, _D), lambda i: (i, 0)),
            pl.BlockSpec((_D, _D), lambda i: (0, 0)),
            pl.BlockSpec((1, _D), lambda i: (0, 0)),
            pl.BlockSpec((1, _D), lambda i: (0, 0)),
            pl.BlockSpec(memory_space=pltpu.SMEM),
        ],
        out_specs=[
            pl.BlockSpec((_BNG, 1), lambda i: (i, 0)),
            pl.BlockSpec(memory_space=pltpu.SMEM),
        ],
        out_shape=[
            jax.ShapeDtypeStruct((_N, 1), jnp.float32),
            jax.ShapeDtypeStruct((1, 1), jnp.float32),
        ],
        scratch_shapes=[pltpu.SMEM((1,), jnp.float32)],
    )(x, w1, b1, w2, b2)


def _sc_body(x_hbm, ids_hbm, gate_hbm, m_hbm,
             num_hbm, den_hbm, bnum_hbm, bden_hbm, bsid_hbm,
             ids_v, gate_v, wv_v, x_v, stage_v, dstage_v, sstage_v,
             zden_v, mi_v, edge_v, m_v,
             zsem, isem0, isem1, gsem0, gsem1, xsem0, xsem1,
             nsem0, nsem1, nsem2, nsem3, dsem0, dsem1, dsem2, dsem3,
             ssem0, ssem1, ssem2, ssem3):
    c = lax.axis_index("c")
    s = lax.axis_index("s")
    w = s * 2 + c
    nblk = jnp.where(w < 21, 98, 97)
    lo = 32 * (97 * w + jnp.minimum(w, 21))
    hi = lo + nblk * _NB

    pltpu.sync_copy(m_hbm, m_v)
    mvec = m_v[...]

    pltpu.sync_copy(
        ids_hbm.at[pl.ds(pl.multiple_of(jnp.maximum(lo - 16, 0), 16), 16)],
        edge_v)
    prev_id = jnp.where(w == 0, -1, edge_v[pl.ds(0, 16)][15])
    pltpu.sync_copy(ids_hbm.at[pl.ds(pl.multiple_of(lo, 16), 16)], edge_v)
    first_id = edge_v[pl.ds(0, 16)][0]
    pltpu.sync_copy(ids_hbm.at[pl.ds(pl.multiple_of(hi - 16, 16), 16)], edge_v)
    b_id = edge_v[pl.ds(0, 16)][15]
    zhi = jnp.where(w == _NW - 1, _S - 1, b_id)

    mi_v[...] = jnp.full((16,), -1, jnp.int32)
    pltpu.sync_copy(mi_v, bsid_hbm.at[pl.ds(pl.multiple_of(w * 32, 16), 16)])
    pltpu.sync_copy(mi_v, bsid_hbm.at[pl.ds(pl.multiple_of(w * 32 + 16, 16), 16)])

    for zi in range(64):
        zden_v[pl.ds(zi * 16, 16)] = jnp.zeros((16,), jnp.float32)
    zcnt = zhi - prev_id
    n64 = zcnt // 64

    def _zf(i, cy):
        pltpu.async_copy(
            zden_v,
            den_hbm.at[pl.ds(pl.multiple_of((prev_id + 1 + i * 64) * 16, 16),
                             1024)],
            zsem)

        @pl.when(i % 8 == 7)
        def _():
            def _w8(r, c2):
                pltpu.make_async_copy(zden_v, den_hbm.at[pl.ds(0, 1024)],
                                      zsem).wait()
                return c2
            lax.fori_loop(0, 8, _w8, 0)
        return cy

    lax.fori_loop(0, n64, _zf, 0)

    def _zrem(r, cy):
        pltpu.make_async_copy(zden_v, den_hbm.at[pl.ds(0, 1024)], zsem).wait()
        return cy

    lax.fori_loop(0, n64 % 8, _zrem, 0)

    @pl.when((zcnt > n64 * 64) & (zcnt >= 64))
    def _():
        pltpu.async_copy(
            zden_v,
            den_hbm.at[pl.ds(pl.multiple_of((zhi - 63) * 16, 16), 1024)],
            zsem).wait()

    @pl.when(zcnt < 64)
    def _():
        def _z1(i, cy):
            pltpu.async_copy(
                zden_v.at[pl.ds(0, 16)],
                den_hbm.at[pl.ds(pl.multiple_of((prev_id + 1 + i) * 16, 16),
                                 16)],
                zsem).wait()
            return cy
        lax.fori_loop(0, zcnt, _z1, 0)

    def _load(t, which):
        blk = lo + t * _NB
        isem, gsem, xsem = ((isem0, gsem0, xsem0) if which == 0
                            else (isem1, gsem1, xsem1))
        ioff = which * _NBP
        pltpu.async_copy(ids_hbm.at[pl.ds(pl.multiple_of(blk, _NB), _NB)],
                         ids_v.at[pl.ds(ioff, _NB)], isem)
        pltpu.async_copy(gate_hbm.at[pl.ds(pl.multiple_of(blk, _NB), _NB)],
                         gate_v.at[pl.ds(ioff, _NB)], gsem)
        pltpu.async_copy(x_hbm.at[pl.ds(pl.multiple_of(blk, _NB), _NB), :],
                         x_v.at[pl.ds(which * _NB, _NB), :], xsem)

    def _wait_load(which):
        isem, gsem, xsem = ((isem0, gsem0, xsem0) if which == 0
                            else (isem1, gsem1, xsem1))
        pltpu.make_async_copy(ids_hbm.at[pl.ds(0, _NB)],
                              ids_v.at[pl.ds(0, _NB)], isem).wait()
        pltpu.make_async_copy(gate_hbm.at[pl.ds(0, _NB)],
                              gate_v.at[pl.ds(0, _NB)], gsem).wait()
        pltpu.make_async_copy(x_hbm.at[pl.ds(0, _NB), :],
                              x_v.at[pl.ds(0, _NB), :], xsem).wait()

    zeros16 = jnp.zeros((16,), jnp.float32)
    zeros32 = tuple(zeros16 for _ in range(32))

    _slotsems = ((nsem0, dsem0, ssem0), (nsem1, dsem1, ssem1),
                 (nsem2, dsem2, ssem2), (nsem3, dsem3, ssem3))

    def _spill_slot(cur_id, spill, den_s, accs, to_last, slot):
        nsem, dsem, ssem = _slotsems[slot]

        @pl.when(spill >= 4)
        def _():
            pltpu.make_async_copy(stage_v.at[pl.ds(0, _D)],
                                  num_hbm.at[pl.ds(0, _D)], nsem).wait()
            pltpu.make_async_copy(dstage_v.at[pl.ds(0, 16)],
                                  den_hbm.at[pl.ds(0, 16)], dsem).wait()
            pltpu.make_async_copy(sstage_v.at[pl.ds(0, 16)],
                                  bsid_hbm.at[pl.ds(0, 16)], ssem).wait()
        base = slot * _D
        for k in range(32):
            stage_v[pl.ds(base + k * 16, 16)] = accs[k]
        rb = slot * 16
        dstage_v[pl.ds(rb, 16)] = jnp.full((16,), 1.0, jnp.float32) * den_s
        sstage_v[pl.ds(rb, 16)] = jnp.full((16,), 1, jnp.int32) * cur_id

        if to_last:
            bslot = w * 2 + 1
            pltpu.async_copy(
                stage_v.at[pl.ds(base, _D)],
                bnum_hbm.at[pl.ds(pl.multiple_of(bslot * _D, _D), _D)], nsem)
            pltpu.async_copy(
                dstage_v.at[pl.ds(rb, 16)],
                bden_hbm.at[pl.ds(pl.multiple_of(bslot * 16, 16), 16)], dsem)
            pltpu.async_copy(
                sstage_v.at[pl.ds(rb, 16)],
                bsid_hbm.at[pl.ds(pl.multiple_of(bslot * 16, 16), 16)], ssem)
        else:
            is_first = cur_id == first_id

            def dma_bnd():
                bslot = w * 2
                pltpu.async_copy(
                    stage_v.at[pl.ds(base, _D)],
                    bnum_hbm.at[pl.ds(pl.multiple_of(bslot * _D, _D), _D)],
                    nsem)
                pltpu.async_copy(
                    dstage_v.at[pl.ds(rb, 16)],
                    bden_hbm.at[pl.ds(pl.multiple_of(bslot * 16, 16), 16)],
                    dsem)
                pltpu.async_copy(
                    sstage_v.at[pl.ds(rb, 16)],
                    bsid_hbm.at[pl.ds(pl.multiple_of(bslot * 16, 16), 16)],
                    ssem)

            def dma_main():
                pltpu.async_copy(
                    stage_v.at[pl.ds(base, _D)],
                    num_hbm.at[pl.ds(pl.multiple_of(cur_id * _D, _D), _D)],
                    nsem)
                pltpu.async_copy(
                    dstage_v.at[pl.ds(rb, 16)],
                    den_hbm.at[pl.ds(pl.multiple_of(cur_id * 16, 16), 16)],
                    dsem)
                pltpu.async_copy(
                    sstage_v.at[pl.ds(rb, 16)],
                    bsid_hbm.at[pl.ds(64 * 16, 16)], ssem)

            lax.cond(is_first, dma_bnd, dma_main)

    def emit_spill(cur_id, spill, den_s, accs, to_last):
        r4 = spill % 4

        def _s(k):
            return lambda: _spill_slot(cur_id, spill, den_s, accs, to_last, k)

        lax.cond(r4 < 2,
                 lambda: lax.cond(r4 == 0, _s(0), _s(1)),
                 lambda: lax.cond(r4 == 2, _s(2), _s(3)))
        return spill + 1

    def _process(t, which, tp, wp, st):
        cur_id, spill, den_s, accs = st
        valid = t < nblk

        @pl.when(valid)
        def _():
            _wait_load(which)

        @pl.when(tp < nblk)
        def _():
            _load(tp, wp)

        ioff = which * _NBP
        for h in range(2):
            wv_v[pl.ds(h * 16, 16)] = jnp.exp(
                gate_v[pl.ds(ioff + h * 16, 16)] - mvec)
        xrow0 = which * _NB

        def inner(j, st2):
            cur_id, spill, den_s, accs = st2
            idj = ids_v[pl.ds(ioff + j, 16)][0]
            wj = wv_v[pl.ds(j, 16)][0]
            is_new = valid & (idj != cur_id)
            do_spill = is_new & (cur_id >= 0)
            spill = lax.cond(
                do_spill,
                lambda: emit_spill(cur_id, spill, den_s, accs, False),
                lambda: spill)
            keep = jnp.where(is_new, 0.0, 1.0)
            wa = jnp.where(valid, wj, 0.0)
            accs = tuple(accs[k] * keep
                         + wa * x_v[xrow0 + j, pl.ds(k * 16, 16)]
                         for k in range(32))
            den_s = den_s * keep + wa
            cur_id = jnp.where(is_new, idj, cur_id)
            return (cur_id, spill, den_s, accs)

        return lax.fori_loop(0, _NB, inner, (cur_id, spill, den_s, accs))

    _load(0, 0)
    st0 = (jnp.int32(-1), jnp.int32(0), jnp.float32(0.0), zeros32)

    def outer2(t2, st):
        st = _process(2 * t2, 0, 2 * t2 + 1, 1, st)
        st = _process(2 * t2 + 1, 1, 2 * t2 + 2, 0, st)
        return st

    cur_f, spill_f, den_f, accs_f = lax.fori_loop(0, _BLKS // 2, outer2, st0)
    spill_l = emit_spill(cur_f, spill_f, den_f, accs_f, True)

    def _drain_slot(slot):
        nsem, dsem, ssem = _slotsems[slot]
        pltpu.make_async_copy(stage_v.at[pl.ds(0, _D)],
                              num_hbm.at[pl.ds(0, _D)], nsem).wait()
        pltpu.make_async_copy(dstage_v.at[pl.ds(0, 16)],
                              den_hbm.at[pl.ds(0, 16)], dsem).wait()
        pltpu.make_async_copy(sstage_v.at[pl.ds(0, 16)],
                              bsid_hbm.at[pl.ds(0, 16)], ssem).wait()

    for _k in range(4):
        @pl.when(spill_l > _k)
        def _(_k=_k):
            _drain_slot(_k)


def _sc_call(x, ids, gate_flat, m16):
    mesh = plsc.VectorSubcoreMesh(core_axis_name="c", subcore_axis_name="s")
    fn = pl.kernel(
        _sc_body,
        out_type=(
            jax.ShapeDtypeStruct((_S * _D,), jnp.float32),
            jax.ShapeDtypeStruct((_S * 16,), jnp.float32),
            jax.ShapeDtypeStruct((64 * _D,), jnp.float32),
            jax.ShapeDtypeStruct((64 * 16,), jnp.float32),
            jax.ShapeDtypeStruct((65 * 16,), jnp.int32),
        ),
        mesh=mesh,
        scratch_types=[
            pltpu.VMEM((2 * _NBP,), jnp.int32),
            pltpu.VMEM((2 * _NBP,), jnp.float32),
            pltpu.VMEM((_NBP,), jnp.float32),
            pltpu.VMEM((2 * _NB, _D), jnp.float32),
            pltpu.VMEM((4 * _D,), jnp.float32),
            pltpu.VMEM((4 * 16,), jnp.float32),
            pltpu.VMEM((4 * 16,), jnp.int32),
            pltpu.VMEM((1024,), jnp.float32),
            pltpu.VMEM((16,), jnp.int32),
            pltpu.VMEM((16,), jnp.int32),
            pltpu.VMEM((16,), jnp.float32),
        ] + [pltpu.SemaphoreType.DMA] * 19,
    )
    return fn(x, ids, gate_flat, m16)


def _div_body(num_ref, den_ref, bnum_ref, bden_ref, bsid_ref, out_ref, dacc):
    i = pl.program_id(0)
    base = i * _BS
    den0 = den_ref[:, 0:1]
    pres = den0 > 0.0
    out_ref[...] = jnp.where(pres, num_ref[...], 0.0)
    dacc[...] = jnp.where(pres, den0, 0.0)
    for e in range(64):
        sid = bsid_ref[e, 0]
        r = sid - base

        @pl.when((sid >= 0) & (r >= 0) & (r < _BS))
        def _():
            out_ref[pl.ds(r, 1), :] += bnum_ref[pl.ds(e, 1), :]
            dacc[pl.ds(r, 1), :] += bden_ref[e, 0]
    d = dacc[...]
    out_ref[...] = jnp.where(d > 0.0, out_ref[...] / (d + 1e-16), 0.0)


def _div_call(num, den, bnum, bden, bsid):
    return pl.pallas_call(
        _div_body,
        grid=(_S // _BS,),
        in_specs=[
            pl.BlockSpec((_BS, _D), lambda i: (i, 0)),
            pl.BlockSpec((_BS, 16), lambda i: (i, 0)),
            pl.BlockSpec((64, _D), lambda i: (0, 0)),
            pl.BlockSpec(memory_space=pltpu.SMEM),
            pl.BlockSpec(memory_space=pltpu.SMEM),
        ],
        out_specs=pl.BlockSpec((_BS, _D), lambda i: (i, 0)),
        out_shape=jax.ShapeDtypeStruct((_S, _D), jnp.float32),
        scratch_shapes=[pltpu.VMEM((_BS, 1), jnp.float32)],
    )(num, den, bnum, bden, bsid)


def kernel(input, bb_id_list, W1, b1, W2, b2):
    x = input.astype(jnp.float32)
    ids = bb_id_list.astype(jnp.int32)
    gate, m = _gate_call(x, W1, b1.reshape(1, _D), W2.reshape(1, _D),
                         b2.reshape(1, 1))
    m16 = jnp.broadcast_to(m.reshape(1), (16,))
    num, den, bnum, bden, bsid = _sc_call(x, ids, gate.reshape(-1), m16)
    return _div_call(num.reshape(_S, _D), den.reshape(_S, 16),
                     bnum.reshape(64, _D), bden.reshape(64, 16),
                     bsid.reshape(65, 16))


# speedup vs baseline: 1.0382x; 1.0382x over previous
import jax
import jax.numpy as jnp
from jax import lax
from jax.experimental import pallas as pl
from jax.experimental.pallas import tpu as pltpu
from jax.experimental.pallas import tpu_sc as plsc

_N, _D, _S = 100000, 512, 20000
_BNG = 4000
_NW = 32
_NB = 32
_NBP = 48
_BLKS = 98
_BS = 2000
_NEG = -3.0e38


def _gate_body(x_ref, w1_ref, b1_ref, w2_ref, b2_ref, gate_ref, m_ref, macc):
    i = pl.program_id(0)
    h = jnp.maximum(
        jnp.dot(x_ref[...], w1_ref[...], preferred_element_type=jnp.float32)
        + b1_ref[...], 0.0)
    g = jnp.sum(h * w2_ref[...], axis=1, keepdims=True) + b2_ref[0, 0]
    gate_ref[...] = g
    bm = jnp.max(g)
    prev = jnp.where(i == 0, _NEG, macc[0])
    cur = jnp.maximum(prev, bm)
    macc[0] = cur

    @pl.when(i == pl.num_programs(0) - 1)
    def _():
        m_ref[0, 0] = cur


def _gate_call(x, w1, b1, w2, b2):
    return pl.pallas_call(
        _gate_body,
        grid=(_N // _BNG,),
        in_specs=[
            pl.BlockSpec((_BNG, _D), lambda i: (i, 0)),
            pl.BlockSpec((_D, _D), lambda i: (0, 0)),
            pl.BlockSpec((1, _D), lambda i: (0, 0)),
            pl.BlockSpec((1, _D), lambda i: (0, 0)),
            pl.BlockSpec(memory_space=pltpu.SMEM),
        ],
        out_specs=[
            pl.BlockSpec((_BNG, 1), lambda i: (i, 0)),
            pl.BlockSpec(memory_space=pltpu.SMEM),
        ],
        out_shape=[
            jax.ShapeDtypeStruct((_N, 1), jnp.float32),
            jax.ShapeDtypeStruct((1, 1), jnp.float32),
        ],
        scratch_shapes=[pltpu.SMEM((1,), jnp.float32)],
    )(x, w1, b1, w2, b2)


def _sc_body(x_hbm, ids_hbm, gate_hbm, m_hbm,
             num_hbm, den_hbm, bnum_hbm, bden_hbm, bsid_hbm,
             ids_v, gate_v, wv_v, x_v, stage_v, dstage_v, sstage_v,
             zden_v, mi_v, edge_v, m_v,
             zsem, isem0, isem1, gsem0, gsem1, xsem0, xsem1,
             nsem0, nsem1, nsem2, nsem3, dsem0, dsem1, dsem2, dsem3,
             ssem0, ssem1, ssem2, ssem3):
    c = lax.axis_index("c")
    s = lax.axis_index("s")
    w = s * 2 + c
    nblk = jnp.where(w < 21, 98, 97)
    lo = 32 * (97 * w + jnp.minimum(w, 21))
    hi = lo + nblk * _NB

    pltpu.sync_copy(m_hbm, m_v)
    mvec = m_v[...]

    pltpu.sync_copy(
        ids_hbm.at[pl.ds(pl.multiple_of(jnp.maximum(lo - 16, 0), 16), 16)],
        edge_v)
    prev_id = jnp.where(w == 0, -1, edge_v[pl.ds(0, 16)][15])
    pltpu.sync_copy(ids_hbm.at[pl.ds(pl.multiple_of(lo, 16), 16)], edge_v)
    first_id = edge_v[pl.ds(0, 16)][0]
    pltpu.sync_copy(ids_hbm.at[pl.ds(pl.multiple_of(hi - 16, 16), 16)], edge_v)
    b_id = edge_v[pl.ds(0, 16)][15]
    zhi = jnp.where(w == _NW - 1, _S - 1, b_id)

    mi_v[...] = jnp.full((16,), -1, jnp.int32)
    pltpu.sync_copy(mi_v, bsid_hbm.at[pl.ds(pl.multiple_of(w * 32, 16), 16)])
    pltpu.sync_copy(mi_v, bsid_hbm.at[pl.ds(pl.multiple_of(w * 32 + 16, 16), 16)])

    for zi in range(64):
        zden_v[pl.ds(zi * 16, 16)] = jnp.zeros((16,), jnp.float32)
    zcnt = zhi - prev_id
    n64 = zcnt // 64

    def _zf(i, cy):
        pltpu.async_copy(
            zden_v,
            den_hbm.at[pl.ds(pl.multiple_of((prev_id + 1 + i * 64) * 16, 16),
                             1024)],
            zsem)

        @pl.when(i % 8 == 7)
        def _():
            def _w8(r, c2):
                pltpu.make_async_copy(zden_v, den_hbm.at[pl.ds(0, 1024)],
                                      zsem).wait()
                return c2
            lax.fori_loop(0, 8, _w8, 0)
        return cy

    lax.fori_loop(0, n64, _zf, 0)

    def _zrem(r, cy):
        pltpu.make_async_copy(zden_v, den_hbm.at[pl.ds(0, 1024)], zsem).wait()
        return cy

    lax.fori_loop(0, n64 % 8, _zrem, 0)

    @pl.when((zcnt > n64 * 64) & (zcnt >= 64))
    def _():
        pltpu.async_copy(
            zden_v,
            den_hbm.at[pl.ds(pl.multiple_of((zhi - 63) * 16, 16), 1024)],
            zsem).wait()

    @pl.when(zcnt < 64)
    def _():
        def _z1(i, cy):
            pltpu.async_copy(
                zden_v.at[pl.ds(0, 16)],
                den_hbm.at[pl.ds(pl.multiple_of((prev_id + 1 + i) * 16, 16),
                                 16)],
                zsem).wait()
            return cy
        lax.fori_loop(0, zcnt, _z1, 0)

    def _load(t, which):
        blk = lo + t * _NB
        isem, gsem, xsem = ((isem0, gsem0, xsem0) if which == 0
                            else (isem1, gsem1, xsem1))
        ioff = which * _NBP
        pltpu.async_copy(ids_hbm.at[pl.ds(pl.multiple_of(blk, _NB), _NB)],
                         ids_v.at[pl.ds(ioff, _NB)], isem)
        pltpu.async_copy(gate_hbm.at[pl.ds(pl.multiple_of(blk, _NB), _NB)],
                         gate_v.at[pl.ds(ioff, _NB)], gsem)
        pltpu.async_copy(x_hbm.at[pl.ds(pl.multiple_of(blk, _NB), _NB), :],
                         x_v.at[pl.ds(which * _NB, _NB), :], xsem)

    def _wait_load(which):
        isem, gsem, xsem = ((isem0, gsem0, xsem0) if which == 0
                            else (isem1, gsem1, xsem1))
        pltpu.make_async_copy(ids_hbm.at[pl.ds(0, _NB)],
                              ids_v.at[pl.ds(0, _NB)], isem).wait()
        pltpu.make_async_copy(gate_hbm.at[pl.ds(0, _NB)],
                              gate_v.at[pl.ds(0, _NB)], gsem).wait()
        pltpu.make_async_copy(x_hbm.at[pl.ds(0, _NB), :],
                              x_v.at[pl.ds(0, _NB), :], xsem).wait()

    zeros16 = jnp.zeros((16,), jnp.float32)
    zeros32 = tuple(zeros16 for _ in range(32))

    _slotsems = ((nsem0, dsem0, ssem0), (nsem1, dsem1, ssem1),
                 (nsem2, dsem2, ssem2), (nsem3, dsem3, ssem3))

    def _spill_slot(cur_id, spill, den_s, accs, to_last, slot):
        nsem, dsem, ssem = _slotsems[slot]

        @pl.when(spill >= 4)
        def _():
            pltpu.make_async_copy(stage_v.at[pl.ds(0, _D)],
                                  num_hbm.at[pl.ds(0, _D)], nsem).wait()
            pltpu.make_async_copy(dstage_v.at[pl.ds(0, 16)],
                                  den_hbm.at[pl.ds(0, 16)], dsem).wait()
            pltpu.make_async_copy(sstage_v.at[pl.ds(0, 16)],
                                  bsid_hbm.at[pl.ds(0, 16)], ssem).wait()
        base = slot * _D
        for k in range(32):
            stage_v[pl.ds(base + k * 16, 16)] = accs[k]
        rb = slot * 16
        dstage_v[pl.ds(rb, 16)] = jnp.full((16,), 1.0, jnp.float32) * den_s
        sstage_v[pl.ds(rb, 16)] = jnp.full((16,), 1, jnp.int32) * cur_id

        if to_last:
            bslot = w * 2 + 1
            pltpu.async_copy(
                stage_v.at[pl.ds(base, _D)],
                bnum_hbm.at[pl.ds(pl.multiple_of(bslot * _D, _D), _D)], nsem)
            pltpu.async_copy(
                dstage_v.at[pl.ds(rb, 16)],
                bden_hbm.at[pl.ds(pl.multiple_of(bslot * 16, 16), 16)], dsem)
            pltpu.async_copy(
                sstage_v.at[pl.ds(rb, 16)],
                bsid_hbm.at[pl.ds(pl.multiple_of(bslot * 16, 16), 16)], ssem)
        else:
            is_first = cur_id == first_id

            def dma_bnd():
                bslot = w * 2
                pltpu.async_copy(
                    stage_v.at[pl.ds(base, _D)],
                    bnum_hbm.at[pl.ds(pl.multiple_of(bslot * _D, _D), _D)],
                    nsem)
                pltpu.async_copy(
                    dstage_v.at[pl.ds(rb, 16)],
                    bden_hbm.at[pl.ds(pl.multiple_of(bslot * 16, 16), 16)],
                    dsem)
                pltpu.async_copy(
                    sstage_v.at[pl.ds(rb, 16)],
                    bsid_hbm.at[pl.ds(pl.multiple_of(bslot * 16, 16), 16)],
                    ssem)

            def dma_main():
                pltpu.async_copy(
                    stage_v.at[pl.ds(base, _D)],
                    num_hbm.at[pl.ds(pl.multiple_of(cur_id * _D, _D), _D)],
                    nsem)
                pltpu.async_copy(
                    dstage_v.at[pl.ds(rb, 16)],
                    den_hbm.at[pl.ds(pl.multiple_of(cur_id * 16, 16), 16)],
                    dsem)
                pltpu.async_copy(
                    sstage_v.at[pl.ds(rb, 16)],
                    bsid_hbm.at[pl.ds(64 * 16, 16)], ssem)

            lax.cond(is_first, dma_bnd, dma_main)

    def emit_spill(cur_id, spill, den_s, accs, to_last):
        r4 = spill % 4

        def _s(k):
            return lambda: _spill_slot(cur_id, spill, den_s, accs, to_last, k)

        lax.cond(r4 < 2,
                 lambda: lax.cond(r4 == 0, _s(0), _s(1)),
                 lambda: lax.cond(r4 == 2, _s(2), _s(3)))
        return spill + 1

    def _process(t, which, tp, wp, st):
        cur_id, spill, den_s, accs = st
        valid = t < nblk

        @pl.when(valid)
        def _():
            _wait_load(which)

        @pl.when(tp < nblk)
        def _():
            _load(tp, wp)

        ioff = which * _NBP
        for h in range(2):
            wv_v[pl.ds(h * 16, 16)] = jnp.exp(
                gate_v[pl.ds(ioff + h * 16, 16)] - mvec)
        xrow0 = which * _NB

        def inner(j, st2):
            cur_id, spill, den_s, accs = st2
            idj = ids_v[pl.ds(ioff + j, 16)][0]
            wj = wv_v[pl.ds(j, 16)][0]
            is_new = valid & (idj != cur_id)
            do_spill = is_new & (cur_id >= 0)
            spill = lax.cond(
                do_spill,
                lambda: emit_spill(cur_id, spill, den_s, accs, False),
                lambda: spill)
            keep = jnp.where(is_new, 0.0, 1.0)
            wa = jnp.where(valid, wj, 0.0)
            accs = tuple(accs[k] * keep
                         + wa * x_v[xrow0 + j, pl.ds(k * 16, 16)]
                         for k in range(32))
            den_s = den_s * keep + wa
            cur_id = jnp.where(is_new, idj, cur_id)
            return (cur_id, spill, den_s, accs)

        return lax.fori_loop(0, _NB, inner, (cur_id, spill, den_s, accs))

    _load(0, 0)
    st0 = (jnp.int32(-1), jnp.int32(0), jnp.float32(0.0), zeros32)

    def outer2(t2, st):
        st = _process(2 * t2, 0, 2 * t2 + 1, 1, st)
        st = _process(2 * t2 + 1, 1, 2 * t2 + 2, 0, st)
        return st

    cur_f, spill_f, den_f, accs_f = lax.fori_loop(0, _BLKS // 2, outer2, st0)
    spill_l = emit_spill(cur_f, spill_f, den_f, accs_f, True)

    def _drain_slot(slot):
        nsem, dsem, ssem = _slotsems[slot]
        pltpu.make_async_copy(stage_v.at[pl.ds(0, _D)],
                              num_hbm.at[pl.ds(0, _D)], nsem).wait()
        pltpu.make_async_copy(dstage_v.at[pl.ds(0, 16)],
                              den_hbm.at[pl.ds(0, 16)], dsem).wait()
        pltpu.make_async_copy(sstage_v.at[pl.ds(0, 16)],
                              bsid_hbm.at[pl.ds(0, 16)], ssem).wait()

    for _k in range(4):
        @pl.when(spill_l > _k)
        def _(_k=_k):
            _drain_slot(_k)


def _sc_call(x, ids, gate_flat, m16):
    mesh = plsc.VectorSubcoreMesh(core_axis_name="c", subcore_axis_name="s")
    fn = pl.kernel(
        _sc_body,
        out_type=(
            jax.ShapeDtypeStruct((_S * _D,), jnp.float32),
            jax.ShapeDtypeStruct((_S * 16,), jnp.float32),
            jax.ShapeDtypeStruct((64 * _D,), jnp.float32),
            jax.ShapeDtypeStruct((64 * 16,), jnp.float32),
            jax.ShapeDtypeStruct((65 * 16,), jnp.int32),
        ),
        mesh=mesh,
        scratch_types=[
            pltpu.VMEM((2 * _NBP,), jnp.int32),
            pltpu.VMEM((2 * _NBP,), jnp.float32),
            pltpu.VMEM((_NBP,), jnp.float32),
            pltpu.VMEM((2 * _NB, _D), jnp.float32),
            pltpu.VMEM((4 * _D,), jnp.float32),
            pltpu.VMEM((4 * 16,), jnp.float32),
            pltpu.VMEM((4 * 16,), jnp.int32),
            pltpu.VMEM((1024,), jnp.float32),
            pltpu.VMEM((16,), jnp.int32),
            pltpu.VMEM((16,), jnp.int32),
            pltpu.VMEM((16,), jnp.float32),
        ] + [pltpu.SemaphoreType.DMA] * 19,
    )
    return fn(x, ids, gate_flat, m16)


def _div_body(num_ref, den_ref, bnum_ref, bden_ref, bsid_ref, out_ref, dacc):
    i = pl.program_id(0)
    base = i * _BS
    den0 = den_ref[:, 0:1]
    pres = den0 > 0.0
    out_ref[...] = jnp.where(pres, num_ref[...], 0.0)
    dacc[...] = jnp.where(pres, den0, 0.0)
    for e in range(64):
        sid = bsid_ref[e, 0]
        r = sid - base

        @pl.when((sid >= 0) & (r >= 0) & (r < _BS))
        def _():
            out_ref[pl.ds(r, 1), :] += bnum_ref[pl.ds(e, 1), :]
            dacc[pl.ds(r, 1), :] += bden_ref[e, 0]
    d = dacc[...]
    out_ref[...] = jnp.where(d > 0.0, out_ref[...] / (d + 1e-16), 0.0)


def _div_call(num, den, bnum, bden, bsid):
    return pl.pallas_call(
        _div_body,
        grid=(_S // _BS,),
        in_specs=[
            pl.BlockSpec((_BS, _D), lambda i: (i, 0)),
            pl.BlockSpec((_BS, 16), lambda i: (i, 0)),
            pl.BlockSpec((64, _D), lambda i: (0, 0)),
            pl.BlockSpec(memory_space=pltpu.SMEM),
            pl.BlockSpec(memory_space=pltpu.SMEM),
        ],
        out_specs=pl.BlockSpec((_BS, _D), lambda i: (i, 0)),
        out_shape=jax.ShapeDtypeStruct((_S, _D), jnp.float32),
        scratch_shapes=[pltpu.VMEM((_BS, 1), jnp.float32)],
    )(num, den, bnum, bden, bsid)


def kernel(input, bb_id_list, W1, b1, W2, b2):
    x = input.astype(jnp.float32)
    ids = bb_id_list.astype(jnp.int32)
    gate, m = _gate_call(x, W1, b1.reshape(1, _D), W2.reshape(1, _D),
                         b2.reshape(1, 1))
    m16 = jnp.broadcast_to(m.reshape(1), (16,))
    num, den, bnum, bden, bsid = _sc_call(x, ids, gate.reshape(-1), m16)
    return _div_call(num.reshape(_S, _D), den.reshape(_S, 16),
                     bnum.reshape(64, _D), bden.reshape(64, 16),
                     bsid.reshape(65, 16))


# speedup vs baseline: 1.0485x; 1.0100x over previous
import jax
import jax.numpy as jnp
from jax import lax
from jax.experimental import pallas as pl
from jax.experimental.pallas import tpu as pltpu
from jax.experimental.pallas import tpu_sc as plsc

_N, _D, _S = 100000, 512, 20000
_BNG = 5000
_NW = 32
_NB = 32
_NBP = 48
_BLKS = 98
_BS = 2000
_NEG = -3.0e38


def _gate_body(x_ref, w1_ref, b1_ref, w2_ref, b2_ref, gate_ref, m_ref, macc):
    i = pl.program_id(0)
    h = jnp.maximum(
        jnp.dot(x_ref[...], w1_ref[...], preferred_element_type=jnp.float32)
        + b1_ref[...], 0.0)
    g = jnp.sum(h * w2_ref[...], axis=1, keepdims=True) + b2_ref[0, 0]
    gate_ref[...] = g
    bm = jnp.max(g)
    prev = jnp.where(i == 0, _NEG, macc[0])
    cur = jnp.maximum(prev, bm)
    macc[0] = cur

    @pl.when(i == pl.num_programs(0) - 1)
    def _():
        m_ref[0, 0] = cur


def _gate_call(x, w1, b1, w2, b2):
    return pl.pallas_call(
        _gate_body,
        grid=(_N // _BNG,),
        in_specs=[
            pl.BlockSpec((_BNG, _D), lambda i: (i, 0)),
            pl.BlockSpec((_D, _D), lambda i: (0, 0)),
            pl.BlockSpec((1, _D), lambda i: (0, 0)),
            pl.BlockSpec((1, _D), lambda i: (0, 0)),
            pl.BlockSpec(memory_space=pltpu.SMEM),
        ],
        out_specs=[
            pl.BlockSpec((_BNG, 1), lambda i: (i, 0)),
            pl.BlockSpec(memory_space=pltpu.SMEM),
        ],
        out_shape=[
            jax.ShapeDtypeStruct((_N, 1), jnp.float32),
            jax.ShapeDtypeStruct((1, 1), jnp.float32),
        ],
        scratch_shapes=[pltpu.SMEM((1,), jnp.float32)],
    )(x, w1, b1, w2, b2)


def _sc_body(x_hbm, ids_hbm, gate_hbm, m_hbm,
             num_hbm, den_hbm, bnum_hbm, bden_hbm, bsid_hbm,
             ids_v, gate_v, wv_v, x_v, stage_v, dstage_v, sstage_v,
             zden_v, mi_v, edge_v, m_v,
             zsem, isem0, isem1, gsem0, gsem1, xsem0, xsem1,
             nsem0, nsem1, nsem2, nsem3, dsem0, dsem1, dsem2, dsem3,
             ssem0, ssem1, ssem2, ssem3):
    c = lax.axis_index("c")
    s = lax.axis_index("s")
    w = s * 2 + c
    nblk = jnp.where(w < 21, 98, 97)
    lo = 32 * (97 * w + jnp.minimum(w, 21))
    hi = lo + nblk * _NB

    pltpu.sync_copy(m_hbm, m_v)
    mvec = m_v[...]

    pltpu.sync_copy(
        ids_hbm.at[pl.ds(pl.multiple_of(jnp.maximum(lo - 16, 0), 16), 16)],
        edge_v)
    prev_id = jnp.where(w == 0, -1, edge_v[pl.ds(0, 16)][15])
    pltpu.sync_copy(ids_hbm.at[pl.ds(pl.multiple_of(lo, 16), 16)], edge_v)
    first_id = edge_v[pl.ds(0, 16)][0]
    pltpu.sync_copy(ids_hbm.at[pl.ds(pl.multiple_of(hi - 16, 16), 16)], edge_v)
    b_id = edge_v[pl.ds(0, 16)][15]
    zhi = jnp.where(w == _NW - 1, _S - 1, b_id)

    mi_v[...] = jnp.full((16,), -1, jnp.int32)
    pltpu.sync_copy(mi_v, bsid_hbm.at[pl.ds(pl.multiple_of(w * 32, 16), 16)])
    pltpu.sync_copy(mi_v, bsid_hbm.at[pl.ds(pl.multiple_of(w * 32 + 16, 16), 16)])

    for zi in range(64):
        zden_v[pl.ds(zi * 16, 16)] = jnp.zeros((16,), jnp.float32)
    zcnt = zhi - prev_id
    n64 = zcnt // 64

    def _zf(i, cy):
        pltpu.async_copy(
            zden_v,
            den_hbm.at[pl.ds(pl.multiple_of((prev_id + 1 + i * 64) * 16, 16),
                             1024)],
            zsem)

        @pl.when(i % 8 == 7)
        def _():
            def _w8(r, c2):
                pltpu.make_async_copy(zden_v, den_hbm.at[pl.ds(0, 1024)],
                                      zsem).wait()
                return c2
            lax.fori_loop(0, 8, _w8, 0)
        return cy

    lax.fori_loop(0, n64, _zf, 0)

    def _zrem(r, cy):
        pltpu.make_async_copy(zden_v, den_hbm.at[pl.ds(0, 1024)], zsem).wait()
        return cy

    lax.fori_loop(0, n64 % 8, _zrem, 0)

    @pl.when((zcnt > n64 * 64) & (zcnt >= 64))
    def _():
        pltpu.async_copy(
            zden_v,
            den_hbm.at[pl.ds(pl.multiple_of((zhi - 63) * 16, 16), 1024)],
            zsem).wait()

    @pl.when(zcnt < 64)
    def _():
        def _z1(i, cy):
            pltpu.async_copy(
                zden_v.at[pl.ds(0, 16)],
                den_hbm.at[pl.ds(pl.multiple_of((prev_id + 1 + i) * 16, 16),
                                 16)],
                zsem).wait()
            return cy
        lax.fori_loop(0, zcnt, _z1, 0)

    def _load(t, which):
        blk = lo + t * _NB
        isem, gsem, xsem = ((isem0, gsem0, xsem0) if which == 0
                            else (isem1, gsem1, xsem1))
        ioff = which * _NBP
        pltpu.async_copy(ids_hbm.at[pl.ds(pl.multiple_of(blk, _NB), _NB)],
                         ids_v.at[pl.ds(ioff, _NB)], isem)
        pltpu.async_copy(gate_hbm.at[pl.ds(pl.multiple_of(blk, _NB), _NB)],
                         gate_v.at[pl.ds(ioff, _NB)], gsem)
        pltpu.async_copy(x_hbm.at[pl.ds(pl.multiple_of(blk, _NB), _NB), :],
                         x_v.at[pl.ds(which * _NB, _NB), :], xsem)

    def _wait_load(which):
        isem, gsem, xsem = ((isem0, gsem0, xsem0) if which == 0
                            else (isem1, gsem1, xsem1))
        pltpu.make_async_copy(ids_hbm.at[pl.ds(0, _NB)],
                              ids_v.at[pl.ds(0, _NB)], isem).wait()
        pltpu.make_async_copy(gate_hbm.at[pl.ds(0, _NB)],
                              gate_v.at[pl.ds(0, _NB)], gsem).wait()
        pltpu.make_async_copy(x_hbm.at[pl.ds(0, _NB), :],
                              x_v.at[pl.ds(0, _NB), :], xsem).wait()

    zeros16 = jnp.zeros((16,), jnp.float32)
    zeros32 = tuple(zeros16 for _ in range(32))

    _slotsems = ((nsem0, dsem0, ssem0), (nsem1, dsem1, ssem1),
                 (nsem2, dsem2, ssem2), (nsem3, dsem3, ssem3))

    def _spill_slot(cur_id, spill, den_s, accs, to_last, slot):
        nsem, dsem, ssem = _slotsems[slot]

        @pl.when(spill >= 4)
        def _():
            pltpu.make_async_copy(stage_v.at[pl.ds(0, _D)],
                                  num_hbm.at[pl.ds(0, _D)], nsem).wait()
            pltpu.make_async_copy(dstage_v.at[pl.ds(0, 16)],
                                  den_hbm.at[pl.ds(0, 16)], dsem).wait()
            pltpu.make_async_copy(sstage_v.at[pl.ds(0, 16)],
                                  bsid_hbm.at[pl.ds(0, 16)], ssem).wait()
        base = slot * _D
        for k in range(32):
            stage_v[pl.ds(base + k * 16, 16)] = accs[k]
        rb = slot * 16
        dstage_v[pl.ds(rb, 16)] = jnp.full((16,), 1.0, jnp.float32) * den_s
        sstage_v[pl.ds(rb, 16)] = jnp.full((16,), 1, jnp.int32) * cur_id

        if to_last:
            bslot = w * 2 + 1
            pltpu.async_copy(
                stage_v.at[pl.ds(base, _D)],
                bnum_hbm.at[pl.ds(pl.multiple_of(bslot * _D, _D), _D)], nsem)
            pltpu.async_copy(
                dstage_v.at[pl.ds(rb, 16)],
                bden_hbm.at[pl.ds(pl.multiple_of(bslot * 16, 16), 16)], dsem)
            pltpu.async_copy(
                sstage_v.at[pl.ds(rb, 16)],
                bsid_hbm.at[pl.ds(pl.multiple_of(bslot * 16, 16), 16)], ssem)
        else:
            is_first = cur_id == first_id

            def dma_bnd():
                bslot = w * 2
                pltpu.async_copy(
                    stage_v.at[pl.ds(base, _D)],
                    bnum_hbm.at[pl.ds(pl.multiple_of(bslot * _D, _D), _D)],
                    nsem)
                pltpu.async_copy(
                    dstage_v.at[pl.ds(rb, 16)],
                    bden_hbm.at[pl.ds(pl.multiple_of(bslot * 16, 16), 16)],
                    dsem)
                pltpu.async_copy(
                    sstage_v.at[pl.ds(rb, 16)],
                    bsid_hbm.at[pl.ds(pl.multiple_of(bslot * 16, 16), 16)],
                    ssem)

            def dma_main():
                pltpu.async_copy(
                    stage_v.at[pl.ds(base, _D)],
                    num_hbm.at[pl.ds(pl.multiple_of(cur_id * _D, _D), _D)],
                    nsem)
                pltpu.async_copy(
                    dstage_v.at[pl.ds(rb, 16)],
                    den_hbm.at[pl.ds(pl.multiple_of(cur_id * 16, 16), 16)],
                    dsem)
                pltpu.async_copy(
                    sstage_v.at[pl.ds(rb, 16)],
                    bsid_hbm.at[pl.ds(64 * 16, 16)], ssem)

            lax.cond(is_first, dma_bnd, dma_main)

    def emit_spill(cur_id, spill, den_s, accs, to_last):
        r4 = spill % 4

        def _s(k):
            return lambda: _spill_slot(cur_id, spill, den_s, accs, to_last, k)

        lax.cond(r4 < 2,
                 lambda: lax.cond(r4 == 0, _s(0), _s(1)),
                 lambda: lax.cond(r4 == 2, _s(2), _s(3)))
        return spill + 1

    def _process(t, which, tp, wp, st):
        cur_id, spill, den_s, accs = st
        valid = t < nblk

        @pl.when(valid)
        def _():
            _wait_load(which)

        @pl.when(tp < nblk)
        def _():
            _load(tp, wp)

        ioff = which * _NBP
        for h in range(2):
            wv_v[pl.ds(h * 16, 16)] = jnp.exp(
                gate_v[pl.ds(ioff + h * 16, 16)] - mvec)
        xrow0 = which * _NB

        def inner(j, st2):
            cur_id, spill, den_s, accs = st2
            idj = ids_v[pl.ds(ioff + j, 16)][0]
            wj = wv_v[pl.ds(j, 16)][0]
            is_new = valid & (idj != cur_id)
            do_spill = is_new & (cur_id >= 0)
            spill = lax.cond(
                do_spill,
                lambda: emit_spill(cur_id, spill, den_s, accs, False),
                lambda: spill)
            keep = jnp.where(is_new, 0.0, 1.0)
            wa = jnp.where(valid, wj, 0.0)
            accs = tuple(accs[k] * keep
                         + wa * x_v[xrow0 + j, pl.ds(k * 16, 16)]
                         for k in range(32))
            den_s = den_s * keep + wa
            cur_id = jnp.where(is_new, idj, cur_id)
            return (cur_id, spill, den_s, accs)

        return lax.fori_loop(0, _NB, inner, (cur_id, spill, den_s, accs))

    _load(0, 0)
    st0 = (jnp.int32(-1), jnp.int32(0), jnp.float32(0.0), zeros32)

    def outer2(t2, st):
        st = _process(2 * t2, 0, 2 * t2 + 1, 1, st)
        st = _process(2 * t2 + 1, 1, 2 * t2 + 2, 0, st)
        return st

    cur_f, spill_f, den_f, accs_f = lax.fori_loop(0, _BLKS // 2, outer2, st0)
    spill_l = emit_spill(cur_f, spill_f, den_f, accs_f, True)

    def _drain_slot(slot):
        nsem, dsem, ssem = _slotsems[slot]
        pltpu.make_async_copy(stage_v.at[pl.ds(0, _D)],
                              num_hbm.at[pl.ds(0, _D)], nsem).wait()
        pltpu.make_async_copy(dstage_v.at[pl.ds(0, 16)],
                              den_hbm.at[pl.ds(0, 16)], dsem).wait()
        pltpu.make_async_copy(sstage_v.at[pl.ds(0, 16)],
                              bsid_hbm.at[pl.ds(0, 16)], ssem).wait()

    for _k in range(4):
        @pl.when(spill_l > _k)
        def _(_k=_k):
            _drain_slot(_k)


def _sc_call(x, ids, gate_flat, m16):
    mesh = plsc.VectorSubcoreMesh(core_axis_name="c", subcore_axis_name="s")
    fn = pl.kernel(
        _sc_body,
        out_type=(
            jax.ShapeDtypeStruct((_S * _D,), jnp.float32),
            jax.ShapeDtypeStruct((_S * 16,), jnp.float32),
            jax.ShapeDtypeStruct((64 * _D,), jnp.float32),
            jax.ShapeDtypeStruct((64 * 16,), jnp.float32),
            jax.ShapeDtypeStruct((65 * 16,), jnp.int32),
        ),
        mesh=mesh,
        scratch_types=[
            pltpu.VMEM((2 * _NBP,), jnp.int32),
            pltpu.VMEM((2 * _NBP,), jnp.float32),
            pltpu.VMEM((_NBP,), jnp.float32),
            pltpu.VMEM((2 * _NB, _D), jnp.float32),
            pltpu.VMEM((4 * _D,), jnp.float32),
            pltpu.VMEM((4 * 16,), jnp.float32),
            pltpu.VMEM((4 * 16,), jnp.int32),
            pltpu.VMEM((1024,), jnp.float32),
            pltpu.VMEM((16,), jnp.int32),
            pltpu.VMEM((16,), jnp.int32),
            pltpu.VMEM((16,), jnp.float32),
        ] + [pltpu.SemaphoreType.DMA] * 19,
    )
    return fn(x, ids, gate_flat, m16)


def _div_body(num_ref, den_ref, bnum_ref, bden_ref, bsid_ref, out_ref, dacc):
    i = pl.program_id(0)
    base = i * _BS
    den0 = den_ref[:, 0:1]
    pres = den0 > 0.0
    out_ref[...] = jnp.where(pres, num_ref[...], 0.0)
    dacc[...] = jnp.where(pres, den0, 0.0)
    for e in range(64):
        sid = bsid_ref[e, 0]
        r = sid - base

        @pl.when((sid >= 0) & (r >= 0) & (r < _BS))
        def _():
            out_ref[pl.ds(r, 1), :] += bnum_ref[pl.ds(e, 1), :]
            dacc[pl.ds(r, 1), :] += bden_ref[e, 0]
    d = dacc[...]
    out_ref[...] = jnp.where(d > 0.0, out_ref[...] / (d + 1e-16), 0.0)


def _div_call(num, den, bnum, bden, bsid):
    return pl.pallas_call(
        _div_body,
        grid=(_S // _BS,),
        in_specs=[
            pl.BlockSpec((_BS, _D), lambda i: (i, 0)),
            pl.BlockSpec((_BS, 16), lambda i: (i, 0)),
            pl.BlockSpec((64, _D), lambda i: (0, 0)),
            pl.BlockSpec(memory_space=pltpu.SMEM),
            pl.BlockSpec(memory_space=pltpu.SMEM),
        ],
        out_specs=pl.BlockSpec((_BS, _D), lambda i: (i, 0)),
        out_shape=jax.ShapeDtypeStruct((_S, _D), jnp.float32),
        scratch_shapes=[pltpu.VMEM((_BS, 1), jnp.float32)],
    )(num, den, bnum, bden, bsid)


def kernel(input, bb_id_list, W1, b1, W2, b2):
    x = input.astype(jnp.float32)
    ids = bb_id_list.astype(jnp.int32)
    gate, m = _gate_call(x, W1, b1.reshape(1, _D), W2.reshape(1, _D),
                         b2.reshape(1, 1))
    m16 = jnp.broadcast_to(m.reshape(1), (16,))
    num, den, bnum, bden, bsid = _sc_call(x, ids, gate.reshape(-1), m16)
    return _div_call(num.reshape(_S, _D), den.reshape(_S, 16),
                     bnum.reshape(64, _D), bden.reshape(64, 16),
                     bsid.reshape(65, 16))
